# baseline (speedup 1.0000x reference)
import functools

import jax
import jax.numpy as jnp
from jax.experimental import pallas as pl
from jax.experimental.pallas import tpu as pltpu


def _shifted2d(x, off, length):
    c, _ = x.shape
    if off < 0:
        zc = jnp.zeros((c, -off), jnp.float32)
        return jnp.concatenate([zc, x[:, : length + off]], axis=1)
    zc = jnp.zeros((c, off), jnp.float32)
    return jnp.concatenate([x[:, off:], zc], axis=1)


def _phased_kernel(x_ref, dw_ref, pw_ref, o_ref, z_scr, st_scr, *,
                   ksize, b_tile, count, eps):
    p = pl.program_id(0)
    b = pl.program_id(1)

    @pl.when(p == 0)
    def _conv():
        @pl.when(b == 0)
        def _init():
            st_scr[...] = jnp.zeros_like(st_scr)

        dw = dw_ref[...]
        pw = pw_ref[...]
        c_in, length = x_ref.shape[1], x_ref.shape[2]
        pad = (ksize - 1) // 2
        s1 = None
        s2 = None
        for i in range(b_tile):
            xi = x_ref[i]
            yi = None
            for k in range(ksize):
                off = k - pad
                tap = xi if off == 0 else _shifted2d(xi, off, length)
                term = tap * dw[:, k].reshape(c_in, 1)
                yi = term if yi is None else yi + term
            z = jnp.dot(pw, yi, preferred_element_type=jnp.float32)
            z_scr[pl.ds(b * b_tile + i, 1)] = z.astype(jnp.bfloat16)[None]
            zs = jnp.sum(z, axis=1, keepdims=True)
            zq = jnp.sum(z * z, axis=1, keepdims=True)
            s1 = zs if s1 is None else s1 + zs
            s2 = zq if s2 is None else s2 + zq
        st_scr[:, 0:1] = st_scr[:, 0:1] + s1
        st_scr[:, 1:2] = st_scr[:, 1:2] + s2

    @pl.when(p == 1)
    def _apply():
        mean = st_scr[:, 0:1] * (1.0 / count)
        var = jnp.maximum(st_scr[:, 1:2] * (1.0 / count) - mean * mean, 0.0)
        inv = jax.lax.rsqrt(var + eps)
        c_out = inv.shape[0]
        scale = inv.reshape(1, c_out, 1)
        shift = (-mean * inv).reshape(1, c_out, 1)
        z = z_scr[pl.ds(b * b_tile, b_tile)].astype(jnp.float32)
        o_ref[...] = jnp.maximum(z * scale + shift, 0.0)


@functools.partial(jax.jit, static_argnames=("ksize", "eps"))
def _fused(x, dw, pw, *, ksize, eps):
    n, c_in, length = x.shape
    c_out = pw.shape[0]
    l_out = length

    b_tile = 8
    nb = n // b_tile

    kfn = functools.partial(_phased_kernel, ksize=ksize, b_tile=b_tile,
                            count=float(n * l_out), eps=eps)
    out = pl.pallas_call(
        kfn,
        grid=(2, nb),
        out_shape=jax.ShapeDtypeStruct((n, c_out, l_out), jnp.float32),
        in_specs=[
            pl.BlockSpec((b_tile, c_in, length),
                         lambda p, b: (b * jnp.where(p == 0, 1, 0), 0, 0)),
            pl.BlockSpec((c_in, ksize), lambda p, b: (0, 0)),
            pl.BlockSpec((c_out, c_in), lambda p, b: (0, 0)),
        ],
        out_specs=pl.BlockSpec((b_tile, c_out, l_out),
                               lambda p, b: (b * jnp.where(p == 1, 1, 0), 0, 0)),
        scratch_shapes=[
            pltpu.VMEM((n, c_out, l_out), jnp.bfloat16),
            pltpu.VMEM((c_out, 8), jnp.float32),
        ],
        compiler_params=pltpu.CompilerParams(
            dimension_semantics=("arbitrary", "arbitrary"),
            vmem_limit_bytes=56 * 1024 * 1024,
        ),
        cost_estimate=pl.CostEstimate(
            flops=n * l_out * (2 * c_in * ksize + 2 * c_out * c_in + 7 * c_out),
            transcendentals=0,
            bytes_accessed=4 * n * c_in * length + 4 * n * c_out * l_out,
        ),
    )(x, dw, pw)
    return out


def kernel(x, dw, db, pw, pb):
    del db, pb
    n, c_in, length = x.shape
    ksize = dw.reshape(c_in, -1).shape[-1]
    c_out = pw.shape[0]
    x = x.astype(jnp.float32)
    dw = dw.astype(jnp.float32).reshape(c_in, ksize)
    pw = pw.astype(jnp.float32).reshape(c_out, c_in)
    return _fused(x, dw, pw, ksize=ksize, eps=1e-5)


# speedup vs baseline: 1.5667x; 1.5667x over previous
import functools

import jax
import jax.numpy as jnp
from jax.experimental import pallas as pl
from jax.experimental.pallas import tpu as pltpu


def _shifted(x, off, length):
    b, c, _ = x.shape
    if off < 0:
        zc = jnp.zeros((b, c, -off), jnp.float32)
        return jnp.concatenate([zc, x[:, :, : length + off]], axis=2)
    zc = jnp.zeros((b, c, off), jnp.float32)
    return jnp.concatenate([x[:, :, off:], zc], axis=2)


def _phased_kernel(x_ref, dw_ref, pw_ref, o_ref, z_scr, st_scr, sc_scr, *,
                   ksize, b_tile, count, eps):
    p = pl.program_id(0)
    b = pl.program_id(1)

    @pl.when(p == 0)
    def _conv():
        @pl.when(b == 0)
        def _init():
            st_scr[...] = jnp.zeros_like(st_scr)

        x = x_ref[...]
        dw = dw_ref[...]
        pw = pw_ref[...]
        _, c_in, length = x.shape
        pad = (ksize - 1) // 2
        y = None
        for k in range(ksize):
            tap = x if k == pad else _shifted(x, k - pad, length)
            term = tap * dw[:, k].reshape(1, c_in, 1)
            y = term if y is None else y + term

        zsum = None
        zsq = None
        zs = []
        for i in range(b_tile):
            z = jnp.dot(pw, y[i], preferred_element_type=jnp.float32)
            zs.append(z.astype(jnp.bfloat16))
            zsum = z if zsum is None else zsum + z
            q = z * z
            zsq = q if zsq is None else zsq + q
        z_scr[pl.ds(b * b_tile, b_tile)] = jnp.stack(zs, axis=0)
        s1 = jnp.sum(zsum, axis=1, keepdims=True)
        s2 = jnp.sum(zsq, axis=1, keepdims=True)
        st_scr[:, 0:1] = st_scr[:, 0:1] + s1
        st_scr[:, 1:2] = st_scr[:, 1:2] + s2

    @pl.when(p == 1)
    def _apply():
        @pl.when(b == 0)
        def _fold():
            mean = st_scr[:, 0:1] * (1.0 / count)
            var = jnp.maximum(st_scr[:, 1:2] * (1.0 / count) - mean * mean, 0.0)
            inv = jax.lax.rsqrt(var + eps)
            sc_scr[:, 0:1] = inv
            sc_scr[:, 1:2] = -mean * inv

        c_out = sc_scr.shape[0]
        scale = sc_scr[:, 0:1].reshape(1, c_out, 1)
        shift = sc_scr[:, 1:2].reshape(1, c_out, 1)
        z = z_scr[pl.ds(b * b_tile, b_tile)].astype(jnp.float32)
        o_ref[...] = jnp.maximum(z * scale + shift, 0.0)


@functools.partial(jax.jit, static_argnames=("ksize", "eps"))
def _fused(x, dw, pw, *, ksize, eps):
    n, c_in, length = x.shape
    c_out = pw.shape[0]
    l_out = length

    b_tile = 8
    nb = n // b_tile

    kfn = functools.partial(_phased_kernel, ksize=ksize, b_tile=b_tile,
                            count=float(n * l_out), eps=eps)
    out = pl.pallas_call(
        kfn,
        grid=(2, nb),
        out_shape=jax.ShapeDtypeStruct((n, c_out, l_out), jnp.float32),
        in_specs=[
            pl.BlockSpec((b_tile, c_in, length),
                         lambda p, b: (b * jnp.where(p == 0, 1, 0), 0, 0)),
            pl.BlockSpec((c_in, ksize), lambda p, b: (0, 0)),
            pl.BlockSpec((c_out, c_in), lambda p, b: (0, 0)),
        ],
        out_specs=pl.BlockSpec((b_tile, c_out, l_out),
                               lambda p, b: (b * jnp.where(p == 1, 1, 0), 0, 0)),
        scratch_shapes=[
            pltpu.VMEM((n, c_out, l_out), jnp.bfloat16),
            pltpu.VMEM((c_out, 8), jnp.float32),
            pltpu.VMEM((c_out, 8), jnp.float32),
        ],
        compiler_params=pltpu.CompilerParams(
            dimension_semantics=("arbitrary", "arbitrary"),
            vmem_limit_bytes=56 * 1024 * 1024,
        ),
        cost_estimate=pl.CostEstimate(
            flops=n * l_out * (2 * c_in * ksize + 2 * c_out * c_in + 7 * c_out),
            transcendentals=0,
            bytes_accessed=4 * n * c_in * length + 4 * n * c_out * l_out,
        ),
    )(x, dw, pw)
    return out


def kernel(x, dw, db, pw, pb):
    del db, pb
    n, c_in, length = x.shape
    ksize = dw.reshape(c_in, -1).shape[-1]
    c_out = pw.shape[0]
    x = x.astype(jnp.float32)
    dw = dw.astype(jnp.float32).reshape(c_in, ksize)
    pw = pw.astype(jnp.float32).reshape(c_out, c_in)
    return _fused(x, dw, pw, ksize=ksize, eps=1e-5)


# speedup vs baseline: 1.7690x; 1.1291x over previous
import functools

import jax
import jax.numpy as jnp
from jax.experimental import pallas as pl
from jax.experimental.pallas import tpu as pltpu


def _shifted(x, off, length):
    b, c, _ = x.shape
    if off < 0:
        zc = jnp.zeros((b, c, -off), jnp.float32)
        return jnp.concatenate([zc, x[:, :, : length + off]], axis=2)
    zc = jnp.zeros((b, c, off), jnp.float32)
    return jnp.concatenate([x[:, :, off:], zc], axis=2)


def _phased_kernel(x_ref, dw_ref, pw_ref, o_ref, z_scr, st_scr, sc_scr, *,
                   ksize, b_tile, count, eps):
    p = pl.program_id(0)
    b = pl.program_id(1)

    @pl.when(p == 0)
    def _conv():
        @pl.when(b == 0)
        def _init():
            st_scr[...] = jnp.zeros_like(st_scr)

        x = x_ref[...]
        dw = dw_ref[...]
        pw = pw_ref[...]
        _, c_in, length = x.shape
        pad = (ksize - 1) // 2
        y = None
        for k in range(ksize):
            tap = x if k == pad else _shifted(x, k - pad, length)
            term = tap * dw[:, k].reshape(1, c_in, 1)
            y = term if y is None else y + term

        zsum = None
        zsq = None
        zs = []
        for i in range(b_tile):
            z = jnp.dot(pw, y[i], preferred_element_type=jnp.float32)
            zs.append(z.astype(jnp.bfloat16))
            zsum = z if zsum is None else zsum + z
            q = z * z
            zsq = q if zsq is None else zsq + q
        z_scr[pl.ds(b * b_tile, b_tile)] = jnp.stack(zs, axis=0)
        s1 = jnp.sum(zsum, axis=1, keepdims=True)
        s2 = jnp.sum(zsq, axis=1, keepdims=True)
        st_scr[:, 0:1] = st_scr[:, 0:1] + s1
        st_scr[:, 1:2] = st_scr[:, 1:2] + s2

    @pl.when(p == 1)
    def _apply():
        @pl.when(b == 0)
        def _fold():
            mean = st_scr[:, 0:1] * (1.0 / count)
            var = jnp.maximum(st_scr[:, 1:2] * (1.0 / count) - mean * mean, 0.0)
            inv = jax.lax.rsqrt(var + eps)
            sc_scr[:, 0:1] = inv
            sc_scr[:, 1:2] = -mean * inv

        c_out = sc_scr.shape[0]
        scale = sc_scr[:, 0:1].reshape(1, c_out, 1)
        shift = sc_scr[:, 1:2].reshape(1, c_out, 1)
        z = z_scr[pl.ds(b * b_tile, b_tile)].astype(jnp.float32)
        o_ref[...] = jnp.maximum(z * scale + shift, 0.0)


@functools.partial(jax.jit, static_argnames=("ksize", "eps"))
def _fused(x, dw, pw, *, ksize, eps):
    n, c_in, length = x.shape
    c_out = pw.shape[0]
    l_out = length

    b_tile = 16
    nb = n // b_tile

    kfn = functools.partial(_phased_kernel, ksize=ksize, b_tile=b_tile,
                            count=float(n * l_out), eps=eps)
    out = pl.pallas_call(
        kfn,
        grid=(2, nb),
        out_shape=jax.ShapeDtypeStruct((n, c_out, l_out), jnp.float32),
        in_specs=[
            pl.BlockSpec((b_tile, c_in, length),
                         lambda p, b: (b * jnp.where(p == 0, 1, 0), 0, 0)),
            pl.BlockSpec((c_in, ksize), lambda p, b: (0, 0)),
            pl.BlockSpec((c_out, c_in), lambda p, b: (0, 0)),
        ],
        out_specs=pl.BlockSpec((b_tile, c_out, l_out),
                               lambda p, b: (b * jnp.where(p == 1, 1, 0), 0, 0)),
        scratch_shapes=[
            pltpu.VMEM((n, c_out, l_out), jnp.bfloat16),
            pltpu.VMEM((c_out, 8), jnp.float32),
            pltpu.VMEM((c_out, 8), jnp.float32),
        ],
        compiler_params=pltpu.CompilerParams(
            dimension_semantics=("arbitrary", "arbitrary"),
            vmem_limit_bytes=60 * 1024 * 1024,
        ),
        cost_estimate=pl.CostEstimate(
            flops=n * l_out * (2 * c_in * ksize + 2 * c_out * c_in + 7 * c_out),
            transcendentals=0,
            bytes_accessed=4 * n * c_in * length + 4 * n * c_out * l_out,
        ),
    )(x, dw, pw)
    return out


def kernel(x, dw, db, pw, pb):
    del db, pb
    n, c_in, length = x.shape
    ksize = dw.reshape(c_in, -1).shape[-1]
    c_out = pw.shape[0]
    x = x.astype(jnp.float32)
    dw = dw.astype(jnp.float32).reshape(c_in, ksize)
    pw = pw.astype(jnp.float32).reshape(c_out, c_in)
    return _fused(x, dw, pw, ksize=ksize, eps=1e-5)


# speedup vs baseline: 1.8829x; 1.0644x over previous
import functools

import jax
import jax.numpy as jnp
from jax.experimental import pallas as pl
from jax.experimental.pallas import tpu as pltpu


def _shifted2(x, off, length):
    c, _ = x.shape
    if off < 0:
        zc = jnp.zeros((c, -off), jnp.float32)
        return jnp.concatenate([zc, x[:, : length + off]], axis=1)
    zc = jnp.zeros((c, off), jnp.float32)
    return jnp.concatenate([x[:, off:], zc], axis=1)


def _phased_kernel(x_ref, dw_ref, pw_ref, o_ref, z_scr, st_scr, sc_scr, *,
                   ksize, b_tile, count, eps):
    p = pl.program_id(0)
    b = pl.program_id(1)

    @pl.when(p == 0)
    def _conv():
        @pl.when(b == 0)
        def _init():
            st_scr[...] = jnp.zeros_like(st_scr)

        dw = dw_ref[...]
        pw = pw_ref[...]
        c_in, length = x_ref.shape[1], x_ref.shape[2]
        pad = (ksize - 1) // 2
        ds = [dw[:, k].reshape(c_in, 1) for k in range(ksize)]

        zsum = None
        zsq = None
        zs = []
        for i in range(b_tile):
            xi = x_ref[i]
            yi = None
            for k in range(ksize):
                tap = xi if k == pad else _shifted2(xi, k - pad, length)
                term = tap * ds[k]
                yi = term if yi is None else yi + term
            z = jnp.dot(pw, yi, preferred_element_type=jnp.float32)
            zs.append(z.astype(jnp.bfloat16))
            zsum = z if zsum is None else zsum + z
            q = z * z
            zsq = q if zsq is None else zsq + q
        z_scr[pl.ds(b * b_tile, b_tile)] = jnp.stack(zs, axis=0)
        s1 = jnp.sum(zsum, axis=1, keepdims=True)
        s2 = jnp.sum(zsq, axis=1, keepdims=True)
        st_scr[:, 0:1] = st_scr[:, 0:1] + s1
        st_scr[:, 1:2] = st_scr[:, 1:2] + s2

    @pl.when(p == 1)
    def _apply():
        @pl.when(b == 0)
        def _fold():
            mean = st_scr[:, 0:1] * (1.0 / count)
            var = jnp.maximum(st_scr[:, 1:2] * (1.0 / count) - mean * mean, 0.0)
            inv = jax.lax.rsqrt(var + eps)
            sc_scr[:, 0:1] = inv
            sc_scr[:, 1:2] = -mean * inv

        c_out = sc_scr.shape[0]
        scale = sc_scr[:, 0:1].reshape(1, c_out, 1)
        shift = sc_scr[:, 1:2].reshape(1, c_out, 1)
        z = z_scr[pl.ds(b * b_tile, b_tile)].astype(jnp.float32)
        o_ref[...] = jnp.maximum(z * scale + shift, 0.0)


@functools.partial(jax.jit, static_argnames=("ksize", "eps"))
def _fused(x, dw, pw, *, ksize, eps):
    n, c_in, length = x.shape
    c_out = pw.shape[0]
    l_out = length

    b_tile = 16
    nb = n // b_tile

    kfn = functools.partial(_phased_kernel, ksize=ksize, b_tile=b_tile,
                            count=float(n * l_out), eps=eps)
    out = pl.pallas_call(
        kfn,
        grid=(2, nb),
        out_shape=jax.ShapeDtypeStruct((n, c_out, l_out), jnp.float32),
        in_specs=[
            pl.BlockSpec((b_tile, c_in, length),
                         lambda p, b: (b * jnp.where(p == 0, 1, 0), 0, 0)),
            pl.BlockSpec((c_in, ksize), lambda p, b: (0, 0)),
            pl.BlockSpec((c_out, c_in), lambda p, b: (0, 0)),
        ],
        out_specs=pl.BlockSpec((b_tile, c_out, l_out),
                               lambda p, b: (b * jnp.where(p == 1, 1, 0), 0, 0)),
        scratch_shapes=[
            pltpu.VMEM((n, c_out, l_out), jnp.bfloat16),
            pltpu.VMEM((c_out, 8), jnp.float32),
            pltpu.VMEM((c_out, 8), jnp.float32),
        ],
        compiler_params=pltpu.CompilerParams(
            dimension_semantics=("arbitrary", "arbitrary"),
            vmem_limit_bytes=60 * 1024 * 1024,
        ),
        cost_estimate=pl.CostEstimate(
            flops=n * l_out * (2 * c_in * ksize + 2 * c_out * c_in + 7 * c_out),
            transcendentals=0,
            bytes_accessed=4 * n * c_in * length + 4 * n * c_out * l_out,
        ),
    )(x, dw, pw)
    return out


def kernel(x, dw, db, pw, pb):
    del db, pb
    n, c_in, length = x.shape
    ksize = dw.reshape(c_in, -1).shape[-1]
    c_out = pw.shape[0]
    x = x.astype(jnp.float32)
    dw = dw.astype(jnp.float32).reshape(c_in, ksize)
    pw = pw.astype(jnp.float32).reshape(c_out, c_in)
    return _fused(x, dw, pw, ksize=ksize, eps=1e-5)
